# baseline (speedup 1.0000x reference)
import functools

import jax
import jax.numpy as jnp
from jax import lax
from jax.experimental import pallas as pl
from jax.experimental.pallas import tpu as pltpu
from jax.experimental.pallas import tpu_sc as plsc

_EDGES = [
    (0, 1), (20, 1), (2, 20), (4, 20), (8, 20), (12, 0), (16, 0), (3, 2),
    (5, 4), (9, 8), (13, 12), (17, 16), (6, 5), (10, 9), (14, 13), (18, 17),
    (7, 6), (11, 10), (15, 14), (19, 18), (21, 7), (22, 7), (23, 11), (24, 11),
]

_NJ = 25
_T = 300
_L = 16
_NW = 32
_COLS = tuple(range(0, _T - _L, _L)) + (_T - _L,)


def _body(bone_hbm, center_hbm, out_hbm, ibuf, obuf, cbuf):
    wid = lax.axis_index("s") * 2 + lax.axis_index("c")
    nb = bone_hbm.shape[0]
    nch = bone_hbm.shape[1]
    per_w = nb // _NW
    base = wid * per_w

    def chunk(g, _):
        b = base + g
        pltpu.sync_copy(bone_hbm.at[pl.ds(b, 1)], ibuf)
        pltpu.sync_copy(center_hbm.at[pl.ds(b, 1)], cbuf)

        for c in range(nch):
            for col in _COLS:
                cv = cbuf[0, c, pl.ds(col, _L)]
                vals = {1: cv}
                obuf[0, c, 1, pl.ds(col, _L)] = cv
                for v1, v2 in _EDGES:
                    v = ibuf[0, c, v1, pl.ds(col, _L)] + vals[v2]
                    vals[v1] = v
                    obuf[0, c, v1, pl.ds(col, _L)] = v

        pltpu.sync_copy(obuf, out_hbm.at[pl.ds(b, 1)])
        return _

    lax.fori_loop(0, per_w, chunk, None)


def kernel(bone, center):
    b, ch, nj, t = bone.shape

    mesh = plsc.VectorSubcoreMesh(core_axis_name="c", subcore_axis_name="s")
    k = functools.partial(
        pl.kernel,
        out_type=jax.ShapeDtypeStruct((b, ch, nj, t), jnp.float32),
        mesh=mesh,
        compiler_params=pltpu.CompilerParams(use_tc_tiling_on_sc=True),
        scratch_types=[
            pltpu.VMEM((1, ch, nj, t), jnp.float32),
            pltpu.VMEM((1, ch, nj, t), jnp.float32),
            pltpu.VMEM((1, ch, t), jnp.float32),
        ],
    )(_body)
    return k(bone, center)


# speedup vs baseline: 1.2335x; 1.2335x over previous
import functools

import jax
import jax.numpy as jnp
from jax import lax
from jax.experimental import pallas as pl
from jax.experimental.pallas import tpu as pltpu
from jax.experimental.pallas import tpu_sc as plsc

_EDGES = [
    (0, 1), (20, 1), (2, 20), (4, 20), (8, 20), (12, 0), (16, 0), (3, 2),
    (5, 4), (9, 8), (13, 12), (17, 16), (6, 5), (10, 9), (14, 13), (18, 17),
    (7, 6), (11, 10), (15, 14), (19, 18), (21, 7), (22, 7), (23, 11), (24, 11),
]

_NJ = 25
_T = 300
_L = 16
_NW = 32
_NBUF = 2
_COLS = tuple(range(0, _T - _L, _L)) + (_T - _L,)


def _body(bone_hbm, center_hbm, out_hbm,
          ibuf0, ibuf1, obuf, cbuf0, cbuf1, bone_sem, cen_sem, out_sem):
    wid = lax.axis_index("s") * 2 + lax.axis_index("c")
    nb = bone_hbm.shape[0]
    nch = bone_hbm.shape[1]
    per_w = nb // _NW
    base = wid * per_w
    ibufs = (ibuf0, ibuf1)
    cbufs = (cbuf0, cbuf1)

    def in_copies(g, slot):
        bidx = base + g
        return (
            pltpu.make_async_copy(bone_hbm.at[pl.ds(bidx, 1)], ibufs[slot],
                                  bone_sem.at[slot]),
            pltpu.make_async_copy(center_hbm.at[pl.ds(bidx, 1)], cbufs[slot],
                                  cen_sem.at[slot]),
        )

    def out_copy(g, c):
        bidx = base + g
        return pltpu.make_async_copy(
            obuf.at[:, pl.ds(c, 1)],
            out_hbm.at[pl.ds(bidx, 1), pl.ds(c, 1)],
            out_sem.at[c])

    def process(g, slot):
        bone_cp, cen_cp = in_copies(g, slot)
        bone_cp.wait()
        cen_cp.wait()
        ibuf = ibufs[slot]
        cbuf = cbufs[slot]

        for c in range(nch):
            @pl.when(g >= 1)
            def _():
                out_copy(g - 1, c).wait()

            for col in _COLS:
                cv = cbuf[0, c, pl.ds(col, _L)]
                vals = {1: cv}
                obuf[0, c, 1, pl.ds(col, _L)] = cv
                for v1, v2 in _EDGES:
                    v = ibuf[0, c, v1, pl.ds(col, _L)] + vals[v2]
                    vals[v1] = v
                    obuf[0, c, v1, pl.ds(col, _L)] = v

            out_copy(g, c).start()

        @pl.when(g + _NBUF < per_w)
        def _():
            bone_np, cen_np = in_copies(g + _NBUF, slot)
            bone_np.start()
            cen_np.start()

    for slot in range(_NBUF):
        bone_cp, cen_cp = in_copies(slot, slot)
        bone_cp.start()
        cen_cp.start()

    def ring(gi, _):
        for slot in range(_NBUF):
            process(gi * _NBUF + slot, slot)
        return _

    lax.fori_loop(0, per_w // _NBUF, ring, None)

    for c in range(3):
        out_copy(per_w - 1, c).wait()


def kernel(bone, center):
    b, ch, nj, t = bone.shape

    mesh = plsc.VectorSubcoreMesh(core_axis_name="c", subcore_axis_name="s")
    k = functools.partial(
        pl.kernel,
        out_type=jax.ShapeDtypeStruct((b, ch, nj, t), jnp.float32),
        mesh=mesh,
        compiler_params=pltpu.CompilerParams(use_tc_tiling_on_sc=True),
        scratch_types=[
            pltpu.VMEM((1, ch, nj, t), jnp.float32),
            pltpu.VMEM((1, ch, nj, t), jnp.float32),
            pltpu.VMEM((1, ch, nj, t), jnp.float32),
            pltpu.VMEM((1, ch, t), jnp.float32),
            pltpu.VMEM((1, ch, t), jnp.float32),
            pltpu.SemaphoreType.DMA((_NBUF,)),
            pltpu.SemaphoreType.DMA((_NBUF,)),
            pltpu.SemaphoreType.DMA((ch,)),
        ],
    )(_body)
    return k(bone, center)


# speedup vs baseline: 1.2652x; 1.0257x over previous
import functools

import jax
import jax.numpy as jnp
from jax import lax
from jax.experimental import pallas as pl
from jax.experimental.pallas import tpu as pltpu
from jax.experimental.pallas import tpu_sc as plsc

_EDGES = [
    (0, 1), (20, 1), (2, 20), (4, 20), (8, 20), (12, 0), (16, 0), (3, 2),
    (5, 4), (9, 8), (13, 12), (17, 16), (6, 5), (10, 9), (14, 13), (18, 17),
    (7, 6), (11, 10), (15, 14), (19, 18), (21, 7), (22, 7), (23, 11), (24, 11),
]

_NJ = 25
_T = 300
_L = 16
_NW = 32
_NBUF = 3
_COLA = _T - 2 * _L + 4
_COLB = _T - _L
_COLS_MAIN = tuple(range(0, _COLA, _L))


def _body(bone_hbm, center_hbm, out_hbm, buf, cbuf, bone_sem, cen_sem, out_sem):
    wid = lax.axis_index("s") * 2 + lax.axis_index("c")
    nb = bone_hbm.shape[0]
    nch = bone_hbm.shape[1]
    per_w = nb // _NW
    base = wid * per_w

    def in_copies(g, slot):
        bidx = base + g
        return (
            pltpu.make_async_copy(bone_hbm.at[pl.ds(bidx, 1)],
                                  buf.at[pl.ds(slot, 1)], bone_sem.at[slot]),
            pltpu.make_async_copy(center_hbm.at[pl.ds(bidx, 1)],
                                  cbuf.at[pl.ds(slot, 1)], cen_sem.at[slot]),
        )

    def out_copy(g, slot):
        bidx = base + g
        return pltpu.make_async_copy(
            buf.at[pl.ds(slot, 1)], out_hbm.at[pl.ds(bidx, 1)],
            out_sem.at[slot])

    def step(g, _):
        slot = g - (g // _NBUF) * _NBUF
        bone_cp, cen_cp = in_copies(g, slot)
        bone_cp.wait()
        cen_cp.wait()

        for c in range(nch):
            for col in _COLS_MAIN:
                cv = cbuf[slot, c, pl.ds(col, _L)]
                vals = {1: cv}
                buf[slot, c, 1, pl.ds(col, _L)] = cv
                for v1, v2 in _EDGES:
                    v = buf[slot, c, v1, pl.ds(col, _L)] + vals[v2]
                    vals[v1] = v
                    buf[slot, c, v1, pl.ds(col, _L)] = v
            ca = cbuf[slot, c, pl.ds(_COLA, _L)]
            cb = cbuf[slot, c, pl.ds(_COLB, _L)]
            vals = {1: (ca, cb)}
            buf[slot, c, 1, pl.ds(_COLA, _L)] = ca
            buf[slot, c, 1, pl.ds(_COLB, _L)] = cb
            for v1, v2 in _EDGES:
                a = buf[slot, c, v1, pl.ds(_COLA, _L)]
                b = buf[slot, c, v1, pl.ds(_COLB, _L)]
                va = a + vals[v2][0]
                vb = b + vals[v2][1]
                vals[v1] = (va, vb)
                buf[slot, c, v1, pl.ds(_COLA, _L)] = va
                buf[slot, c, v1, pl.ds(_COLB, _L)] = vb

        out_copy(g, slot).start()

        @pl.when(g >= 1)
        def _():
            prev = g - 1
            out_copy(prev, prev - (prev // _NBUF) * _NBUF).wait()

        @pl.when(g + 2 < per_w)
        def _():
            nxt = g + 2
            nslot = nxt - (nxt // _NBUF) * _NBUF
            bone_np, cen_np = in_copies(nxt, nslot)
            bone_np.start()
            cen_np.start()

        return _

    for g0 in range(2):
        bone_cp, cen_cp = in_copies(g0, g0)
        bone_cp.start()
        cen_cp.start()

    lax.fori_loop(0, per_w, step, None)

    last = per_w - 1
    out_copy(last, last - (last // _NBUF) * _NBUF).wait()


def kernel(bone, center):
    b, ch, nj, t = bone.shape

    mesh = plsc.VectorSubcoreMesh(core_axis_name="c", subcore_axis_name="s")
    k = functools.partial(
        pl.kernel,
        out_type=jax.ShapeDtypeStruct((b, ch, nj, t), jnp.float32),
        mesh=mesh,
        compiler_params=pltpu.CompilerParams(use_tc_tiling_on_sc=True),
        scratch_types=[
            pltpu.VMEM((_NBUF, ch, nj, t), jnp.float32),
            pltpu.VMEM((_NBUF, ch, t), jnp.float32),
            pltpu.SemaphoreType.DMA((_NBUF,)),
            pltpu.SemaphoreType.DMA((_NBUF,)),
            pltpu.SemaphoreType.DMA((_NBUF,)),
        ],
    )(_body)
    return k(bone, center)
